# Initial kernel scaffold; baseline (speedup 1.0000x reference)
import jax
import jax.numpy as jnp
from jax.experimental import pallas as pl


def kernel(x, edge_index, W1, as1, ad1, b1, W2, as2, ad2, b2, Wm1, bm1, Wm2, bm2):
    raise NotImplementedError("write your pallas kernel here")


# speedup vs baseline: 33.5514x; 33.5514x over previous
import functools

import jax
import jax.numpy as jnp
from jax import lax
from jax.experimental import pallas as pl
from jax.experimental.pallas import tpu as pltpu
from jax.experimental.pallas import tpu_sc as plsc

_N = 10000
_E = 320000
_DIN = 128
_HEADS = 8
_H1 = 16
_H2 = 64

_NC = 2
_NS = 16
_NW = _NC * _NS
_EPW = _E // _NW
_C = 80
_NCHUNK = _EPW // _C
_NPAD = 10240
_RPT = _NPAD // _NS

_HIGH = lax.Precision.HIGHEST


def _dot(a, b):
    return jnp.dot(a, b, precision=_HIGH, preferred_element_type=jnp.float32)


def _tca_body(x_ref, w1_ref, p1s_ref, p1d_ref, h_ref, as_ref, ad_ref):
    h = _dot(x_ref[...], w1_ref[...])
    h_ref[...] = h
    as_ref[...] = _dot(h, p1s_ref[...])
    ad_ref[...] = _dot(h, p1d_ref[...])


def _tcb_body(acc_ref, den_ref, b1_ref, p_ref, w2_ref, p2s_ref, p2d_ref,
              h2_ref, as_ref, ad_ref):
    dsum = den_ref[0] + den_ref[1]
    dexp = _dot(dsum, p_ref[...])
    acc = acc_ref[0] + acc_ref[1]
    x2 = jnp.maximum(acc / (dexp + 1e-16) + b1_ref[...], 0.0)
    h2 = _dot(x2, w2_ref[...])
    h2_ref[...] = h2
    as_ref[...] = _dot(h2, p2s_ref[...])
    ad_ref[...] = _dot(h2, p2d_ref[...])


def _tcc_body(den_ref, rden_ref):
    r = 1.0 / (den_ref[0] + den_ref[1] + 1e-16)
    rden_ref[...] = jnp.concatenate(
        [r, jnp.zeros((r.shape[0], 112), jnp.float32)], axis=1)


def _tcd_body(acc_ref, b2_ref, wm1_ref, bm1_ref, wm2_ref, bm2_ref, out_ref):
    o2 = (acc_ref[0] + acc_ref[1]) * 0.125 + b2_ref[...]
    x3 = jnp.maximum(_dot(o2, wm1_ref[...]) + bm1_ref[...], 0.0)
    out_ref[...] = _dot(x3, wm2_ref[...]) + bm2_ref[...]


_MESH = plsc.VectorSubcoreMesh(core_axis_name="c", subcore_axis_name="s")


def _scden_body(src_hbm, dst_hbm, as_hbm, ad_hbm, z16_hbm,
                den_out, w_out,
                den_sh, sidx, didx, asb, adb, wb2, wb1,
                sem0, sem1):
    c = lax.axis_index("c")
    s = lax.axis_index("s")
    wid = s * _NC + c
    rb = s * _RPT
    pltpu.sync_copy(z16_hbm.at[pl.ds(rb, _RPT)], den_sh.at[pl.ds(rb, _RPT)])
    plsc.subcore_barrier()

    base = wid * _EPW

    def chunk(i, carry):
        eb = base + i * _C
        pltpu.sync_copy(src_hbm.at[pl.ds(eb, _C)], sidx)
        pltpu.sync_copy(dst_hbm.at[pl.ds(eb, _C)], didx)
        cp0 = pltpu.async_copy(as_hbm.at[sidx], asb, sem0)
        cp1 = pltpu.async_copy(ad_hbm.at[didx], adb, sem1)
        cp0.wait()
        cp1.wait()

        def edge(e, cc):
            av = asb[e, pl.ds(0, 16)] + adb[e, pl.ds(0, 16)]
            w = jnp.exp(jnp.maximum(av, av * 0.2))
            wb2[e, :] = w
            wb1[pl.ds(e * 16, 16)] = w
            return cc

        lax.fori_loop(0, _C, edge, 0)
        pltpu.sync_copy(wb2, den_sh.at[didx], add=True)
        pltpu.sync_copy(wb1, w_out.at[pl.ds(eb * 16, _C * 16)])
        return carry

    lax.fori_loop(0, _NCHUNK, chunk, 0)
    plsc.subcore_barrier()
    pltpu.sync_copy(den_sh.at[pl.ds(rb, _RPT)], den_out.at[c, pl.ds(rb, _RPT)])


_scden = functools.partial(
    pl.kernel, _scden_body, mesh=_MESH,
    out_type=[jax.ShapeDtypeStruct((_NC, _NPAD, 16), jnp.float32),
              jax.ShapeDtypeStruct((_E * 16,), jnp.float32)],
    scratch_types=[
        pltpu.VMEM_SHARED((_NPAD, 16), jnp.float32),
        pltpu.VMEM((_C,), jnp.int32),
        pltpu.VMEM((_C,), jnp.int32),
        pltpu.VMEM((_C, 128), jnp.float32),
        pltpu.VMEM((_C, 128), jnp.float32),
        pltpu.VMEM((_C, 16), jnp.float32),
        pltpu.VMEM((_C * 16,), jnp.float32),
        pltpu.SemaphoreType.DMA,
        pltpu.SemaphoreType.DMA,
    ],
)


def _scmsg1_body(src_hbm, dst_hbm, h1_hbm, w_hbm, z128_hbm,
                 acc_out,
                 acc_sh, sidx, didx, hb, wb1, mb,
                 sem0):
    c = lax.axis_index("c")
    s = lax.axis_index("s")
    wid = s * _NC + c
    rb = s * _RPT
    pltpu.sync_copy(z128_hbm.at[pl.ds(rb, _RPT)], acc_sh.at[pl.ds(rb, _RPT)])
    plsc.subcore_barrier()

    base = wid * _EPW

    def chunk(i, carry):
        eb = base + i * _C
        pltpu.sync_copy(src_hbm.at[pl.ds(eb, _C)], sidx)
        pltpu.sync_copy(dst_hbm.at[pl.ds(eb, _C)], didx)
        pltpu.sync_copy(w_hbm.at[pl.ds(eb * 16, _C * 16)], wb1)
        pltpu.async_copy(h1_hbm.at[sidx], hb, sem0).wait()

        def edge(e, cc):
            w = wb1[pl.ds(e * 16, 16)]
            for k in range(_HEADS):
                mb[e, pl.ds(k * 16, 16)] = hb[e, pl.ds(k * 16, 16)] * w[k]
            return cc

        lax.fori_loop(0, _C, edge, 0)
        pltpu.sync_copy(mb, acc_sh.at[didx], add=True)
        return carry

    lax.fori_loop(0, _NCHUNK, chunk, 0)
    plsc.subcore_barrier()
    pltpu.sync_copy(acc_sh.at[pl.ds(rb, _RPT)], acc_out.at[c, pl.ds(rb, _RPT)])


_scmsg1 = functools.partial(
    pl.kernel, _scmsg1_body, mesh=_MESH,
    out_type=[jax.ShapeDtypeStruct((_NC, _NPAD, 128), jnp.float32)],
    scratch_types=[
        pltpu.VMEM_SHARED((_NPAD, 128), jnp.float32),
        pltpu.VMEM((_C,), jnp.int32),
        pltpu.VMEM((_C,), jnp.int32),
        pltpu.VMEM((_C, 128), jnp.float32),
        pltpu.VMEM((_C * 16,), jnp.float32),
        pltpu.VMEM((_C, 128), jnp.float32),
        pltpu.SemaphoreType.DMA,
    ],
)


def _scmsg2_body(src_hbm, dst_hbm, h2_hbm, w_hbm, rden_hbm, z64_hbm,
                 acc_out,
                 acc_sh, sidx, didx, hb, wb1, rdb, mb,
                 sem0, sem1):
    c = lax.axis_index("c")
    s = lax.axis_index("s")
    wid = s * _NC + c
    rb = s * _RPT
    pltpu.sync_copy(z64_hbm.at[pl.ds(rb, _RPT)], acc_sh.at[pl.ds(rb, _RPT)])
    plsc.subcore_barrier()

    base = wid * _EPW

    def chunk(i, carry):
        eb = base + i * _C
        pltpu.sync_copy(src_hbm.at[pl.ds(eb, _C)], sidx)
        pltpu.sync_copy(dst_hbm.at[pl.ds(eb, _C)], didx)
        pltpu.sync_copy(w_hbm.at[pl.ds(eb * 16, _C * 16)], wb1)
        cp0 = pltpu.async_copy(h2_hbm.at[sidx], hb, sem0)
        cp1 = pltpu.async_copy(rden_hbm.at[didx], rdb, sem1)
        cp0.wait()
        cp1.wait()

        def edge(e, cc):
            a = wb1[pl.ds(e * 16, 16)] * rdb[e, pl.ds(0, 16)]
            m = [jnp.zeros((16,), jnp.float32) for _ in range(4)]
            for k in range(_HEADS):
                avk = a[k]
                for q in range(4):
                    m[q] = m[q] + hb[e, pl.ds(k * 64 + q * 16, 16)] * avk
            for q in range(4):
                mb[e, pl.ds(q * 16, 16)] = m[q]
            return cc

        lax.fori_loop(0, _C, edge, 0)
        pltpu.sync_copy(mb, acc_sh.at[didx], add=True)
        return carry

    lax.fori_loop(0, _NCHUNK, chunk, 0)
    plsc.subcore_barrier()
    pltpu.sync_copy(acc_sh.at[pl.ds(rb, _RPT)], acc_out.at[c, pl.ds(rb, _RPT)])


_scmsg2 = functools.partial(
    pl.kernel, _scmsg2_body, mesh=_MESH,
    out_type=[jax.ShapeDtypeStruct((_NC, _NPAD, 64), jnp.float32)],
    scratch_types=[
        pltpu.VMEM_SHARED((_NPAD, 64), jnp.float32),
        pltpu.VMEM((_C,), jnp.int32),
        pltpu.VMEM((_C,), jnp.int32),
        pltpu.VMEM((_C, 512), jnp.float32),
        pltpu.VMEM((_C * 16,), jnp.float32),
        pltpu.VMEM((_C, 128), jnp.float32),
        pltpu.VMEM((_C, 64), jnp.float32),
        pltpu.SemaphoreType.DMA,
        pltpu.SemaphoreType.DMA,
    ],
)


def kernel(x, edge_index, W1, as1, ad1, b1, W2, as2, ad2, b2, Wm1, bm1, Wm2, bm2):
    f32 = jnp.float32
    src = edge_index[0]
    dst = edge_index[1]

    eye8 = jnp.eye(_HEADS, 128, dtype=f32)
    p1s = (as1[:, :, None] * eye8[:, None, :]).reshape(_HEADS * _H1, 128)
    p1d = (ad1[:, :, None] * eye8[:, None, :]).reshape(_HEADS * _H1, 128)
    p2s = (as2[:, :, None] * eye8[:, None, :]).reshape(_HEADS * _H2, 128)
    p2d = (ad2[:, :, None] * eye8[:, None, :]).reshape(_HEADS * _H2, 128)
    pexp = jnp.concatenate(
        [jnp.kron(jnp.eye(_HEADS, dtype=f32), jnp.ones((1, 16), f32)),
         jnp.zeros((8, 128), f32)], axis=0)

    z128 = jnp.zeros((_NPAD, 128), f32)
    z64 = jnp.zeros((_NPAD, 64), f32)
    z16 = jnp.zeros((_NPAD, 16), f32)

    nb = 10
    rows = _N // nb

    h1, a1s, a1d = pl.pallas_call(
        _tca_body,
        grid=(nb,),
        in_specs=[
            pl.BlockSpec((rows, _DIN), lambda i: (i, 0)),
            pl.BlockSpec((_DIN, 128), lambda i: (0, 0)),
            pl.BlockSpec((128, 128), lambda i: (0, 0)),
            pl.BlockSpec((128, 128), lambda i: (0, 0)),
        ],
        out_specs=[
            pl.BlockSpec((rows, 128), lambda i: (i, 0)),
            pl.BlockSpec((rows, 128), lambda i: (i, 0)),
            pl.BlockSpec((rows, 128), lambda i: (i, 0)),
        ],
        out_shape=[
            jax.ShapeDtypeStruct((_N, 128), f32),
            jax.ShapeDtypeStruct((_N, 128), f32),
            jax.ShapeDtypeStruct((_N, 128), f32),
        ],
    )(x, W1, p1s, p1d)

    den1, w1e = _scden()(src, dst, a1s, a1d, z16)
    (acc1,) = _scmsg1()(src, dst, h1, w1e, z128)

    h2, a2s, a2d = pl.pallas_call(
        _tcb_body,
        grid=(nb,),
        in_specs=[
            pl.BlockSpec((_NC, rows, 128), lambda i: (0, i, 0)),
            pl.BlockSpec((_NC, rows, 16), lambda i: (0, i, 0)),
            pl.BlockSpec((1, 128), lambda i: (0, 0)),
            pl.BlockSpec((16, 128), lambda i: (0, 0)),
            pl.BlockSpec((128, 512), lambda i: (0, 0)),
            pl.BlockSpec((512, 128), lambda i: (0, 0)),
            pl.BlockSpec((512, 128), lambda i: (0, 0)),
        ],
        out_specs=[
            pl.BlockSpec((rows, 512), lambda i: (i, 0)),
            pl.BlockSpec((rows, 128), lambda i: (i, 0)),
            pl.BlockSpec((rows, 128), lambda i: (i, 0)),
        ],
        out_shape=[
            jax.ShapeDtypeStruct((_N, 512), f32),
            jax.ShapeDtypeStruct((_N, 128), f32),
            jax.ShapeDtypeStruct((_N, 128), f32),
        ],
    )(acc1, den1, b1.reshape(1, 128), pexp, W2, p2s, p2d)

    den2, w2e = _scden()(src, dst, a2s, a2d, z16)

    rden = pl.pallas_call(
        _tcc_body,
        grid=(1,),
        in_specs=[pl.BlockSpec((_NC, _N, 16), lambda i: (0, 0, 0))],
        out_specs=pl.BlockSpec((_N, 128), lambda i: (0, 0)),
        out_shape=jax.ShapeDtypeStruct((_N, 128), f32),
    )(den2)

    (acc2,) = _scmsg2()(src, dst, h2, w2e, rden, z64)

    wm2p = jnp.zeros((64, 128), f32).at[:, :2].set(Wm2)
    bm2p = jnp.zeros((1, 128), f32).at[0, :2].set(bm2)
    outp = pl.pallas_call(
        _tcd_body,
        grid=(nb,),
        in_specs=[
            pl.BlockSpec((_NC, rows, 64), lambda i: (0, i, 0)),
            pl.BlockSpec((1, 64), lambda i: (0, 0)),
            pl.BlockSpec((64, 64), lambda i: (0, 0)),
            pl.BlockSpec((1, 64), lambda i: (0, 0)),
            pl.BlockSpec((64, 128), lambda i: (0, 0)),
            pl.BlockSpec((1, 128), lambda i: (0, 0)),
        ],
        out_specs=pl.BlockSpec((rows, 128), lambda i: (i, 0)),
        out_shape=jax.ShapeDtypeStruct((_N, 128), f32),
    )(acc2, b2.reshape(1, 64), Wm1, bm1.reshape(1, 64), wm2p, bm2p)

    return outp[:, :2]
